# baseline (speedup 1.0000x reference)
import functools
import jax
import jax.numpy as jnp
from jax.experimental import pallas as pl
from jax.experimental.pallas import tpu as pltpu

K = 3
NLAT_OUT = 128
NLON = 256
NNZ = 32
CIN = 64
COUT = 64
LAT_BLK = 8
GRID = NLAT_OUT // LAT_BLK


def _disco_kernel(idx_ref, val_ref, xt_ref, w2_ref, bias_ref, out_ref, y_ref):
  for k in range(K):
    for j in range(LAT_BLK):
      def body(n, acc):
        iflat = idx_ref[k, j, n]
        lat = iflat >> 8
        shift = (NLON - (iflat & (NLON - 1))) & (NLON - 1)
        val = val_ref[k, j, n]
        tile = xt_ref[lat]
        return acc + val * pltpu.roll(tile, shift, 1)

      acc = jax.lax.fori_loop(
          0, NNZ, body, jnp.zeros((CIN, NLON), jnp.float32))
      y_ref[k * CIN:(k + 1) * CIN, j * NLON:(j + 1) * NLON] = acc
  out_ref[...] = (
      jnp.dot(w2_ref[...], y_ref[...], preferred_element_type=jnp.float32)
      + bias_ref[...]
  )


@jax.jit
def kernel(x, psi_idx, psi_vals, weight, bias):
  B, Cin, nlat_in, nlon_in = x.shape
  xt = jnp.transpose(x[0], (1, 0, 2))
  idx = psi_idx[2].reshape(K, NLAT_OUT, NNZ)
  vals = psi_vals.reshape(K, NLAT_OUT, NNZ)
  w2 = jnp.transpose(weight, (0, 2, 1)).reshape(COUT, K * CIN)
  bias2 = bias[:, None]

  out2d = pl.pallas_call(
      _disco_kernel,
      grid=(GRID,),
      in_specs=[
          pl.BlockSpec((K, LAT_BLK, NNZ), lambda i: (0, i, 0),
                       memory_space=pltpu.SMEM),
          pl.BlockSpec((K, LAT_BLK, NNZ), lambda i: (0, i, 0),
                       memory_space=pltpu.SMEM),
          pl.BlockSpec((nlat_in, Cin, nlon_in), lambda i: (0, 0, 0)),
          pl.BlockSpec((COUT, K * CIN), lambda i: (0, 0)),
          pl.BlockSpec((COUT, 1), lambda i: (0, 0)),
      ],
      out_specs=pl.BlockSpec((COUT, LAT_BLK * NLON), lambda i: (0, i)),
      out_shape=jax.ShapeDtypeStruct((COUT, NLAT_OUT * NLON), jnp.float32),
      scratch_shapes=[pltpu.VMEM((K * CIN, LAT_BLK * NLON), jnp.float32)],
  )(idx, vals, xt, w2, bias2)

  return out2d.reshape(1, COUT, NLAT_OUT, NLON)


# speedup vs baseline: 2.3898x; 2.3898x over previous
import functools
import jax
import jax.numpy as jnp
from jax.experimental import pallas as pl
from jax.experimental.pallas import tpu as pltpu

K = 3
NLAT_OUT = 128
NLON = 256
NNZ = 32
CIN = 64
COUT = 64
LAT_BLK = 8
GRID = NLAT_OUT // LAT_BLK


def _disco_kernel(idx_ref, val_ref, xt_ref, w2_ref, bias_ref, out_ref, y_ref):
  UNROLL = 4
  for k in range(K):
    for j in range(LAT_BLK):
      def body(n2, accs):
        a0, a1 = accs
        for u in range(UNROLL):
          n = n2 * UNROLL + u
          iflat = idx_ref[k, j, n]
          lat = iflat >> 8
          shift = (NLON - (iflat & (NLON - 1))) & (NLON - 1)
          val = val_ref[k, j, n]
          t = val * pltpu.roll(xt_ref[lat], shift, 1)
          if u % 2 == 0:
            a0 = a0 + t
          else:
            a1 = a1 + t
        return (a0, a1)

      z = jnp.zeros((CIN, NLON), jnp.float32)
      a0, a1 = jax.lax.fori_loop(0, NNZ // UNROLL, body, (z, z))
      y_ref[k * CIN:(k + 1) * CIN, j * NLON:(j + 1) * NLON] = a0 + a1
  out_ref[...] = (
      jnp.dot(w2_ref[...], y_ref[...], preferred_element_type=jnp.float32)
      + bias_ref[...]
  )


@jax.jit
def kernel(x, psi_idx, psi_vals, weight, bias):
  B, Cin, nlat_in, nlon_in = x.shape
  xt = jnp.transpose(x[0], (1, 0, 2))
  idx = psi_idx[2].reshape(K, NLAT_OUT, NNZ)
  vals = psi_vals.reshape(K, NLAT_OUT, NNZ)
  w2 = jnp.transpose(weight, (0, 2, 1)).reshape(COUT, K * CIN)
  bias2 = bias[:, None]

  out2d = pl.pallas_call(
      _disco_kernel,
      grid=(GRID,),
      in_specs=[
          pl.BlockSpec((K, LAT_BLK, NNZ), lambda i: (0, i, 0),
                       memory_space=pltpu.SMEM),
          pl.BlockSpec((K, LAT_BLK, NNZ), lambda i: (0, i, 0),
                       memory_space=pltpu.SMEM),
          pl.BlockSpec((nlat_in, Cin, nlon_in), lambda i: (0, 0, 0)),
          pl.BlockSpec((COUT, K * CIN), lambda i: (0, 0)),
          pl.BlockSpec((COUT, 1), lambda i: (0, 0)),
      ],
      out_specs=pl.BlockSpec((COUT, LAT_BLK * NLON), lambda i: (0, i)),
      out_shape=jax.ShapeDtypeStruct((COUT, NLAT_OUT * NLON), jnp.float32),
      scratch_shapes=[pltpu.VMEM((K * CIN, LAT_BLK * NLON), jnp.float32)],
  )(idx, vals, xt, w2, bias2)

  return out2d.reshape(1, COUT, NLAT_OUT, NLON)


# speedup vs baseline: 3.1315x; 1.3103x over previous
import functools
import jax
import jax.numpy as jnp
from jax.experimental import pallas as pl
from jax.experimental.pallas import tpu as pltpu

K = 3
NLAT_OUT = 128
NLON = 256
NNZ = 32
CIN = 64
COUT = 64
LAT_BLK = 8
GRID = NLAT_OUT // LAT_BLK


def _disco_kernel(idx_ref, val_ref, xt_ref, w2_ref, bias_ref, out_ref, y_ref):
  UNROLL = 8
  for k in range(K):
    for j in range(LAT_BLK):
      def body(n2, accs):
        a0, a1 = accs
        for u in range(UNROLL):
          n = n2 * UNROLL + u
          iflat = idx_ref[k, j, n]
          lat = iflat >> 8
          shift = (NLON - (iflat & (NLON - 1))) & (NLON - 1)
          val = val_ref[k, j, n]
          t = val * pltpu.roll(xt_ref[lat], shift, 1)
          if u % 2 == 0:
            a0 = a0 + t
          else:
            a1 = a1 + t
        return (a0, a1)

      z = jnp.zeros((CIN, NLON), jnp.float32)
      a0, a1 = jax.lax.fori_loop(0, NNZ // UNROLL, body, (z, z))
      y_ref[k * CIN:(k + 1) * CIN, j * NLON:(j + 1) * NLON] = a0 + a1
  out_ref[...] = (
      jnp.dot(w2_ref[...], y_ref[...], preferred_element_type=jnp.float32)
      + bias_ref[...]
  )


@jax.jit
def kernel(x, psi_idx, psi_vals, weight, bias):
  B, Cin, nlat_in, nlon_in = x.shape
  xt = jnp.transpose(x[0], (1, 0, 2))
  idx = psi_idx[2].reshape(K, NLAT_OUT, NNZ)
  vals = psi_vals.reshape(K, NLAT_OUT, NNZ)
  w2 = jnp.transpose(weight, (0, 2, 1)).reshape(COUT, K * CIN)
  bias2 = bias[:, None]

  out2d = pl.pallas_call(
      _disco_kernel,
      grid=(GRID,),
      in_specs=[
          pl.BlockSpec((K, LAT_BLK, NNZ), lambda i: (0, i, 0),
                       memory_space=pltpu.SMEM),
          pl.BlockSpec((K, LAT_BLK, NNZ), lambda i: (0, i, 0),
                       memory_space=pltpu.SMEM),
          pl.BlockSpec((nlat_in, Cin, nlon_in), lambda i: (0, 0, 0)),
          pl.BlockSpec((COUT, K * CIN), lambda i: (0, 0)),
          pl.BlockSpec((COUT, 1), lambda i: (0, 0)),
      ],
      out_specs=pl.BlockSpec((COUT, LAT_BLK * NLON), lambda i: (0, i)),
      out_shape=jax.ShapeDtypeStruct((COUT, NLAT_OUT * NLON), jnp.float32),
      scratch_shapes=[pltpu.VMEM((K * CIN, LAT_BLK * NLON), jnp.float32)],
  )(idx, vals, xt, w2, bias2)

  return out2d.reshape(1, COUT, NLAT_OUT, NLON)
